# Initial kernel scaffold; baseline (speedup 1.0000x reference)
import jax
import jax.numpy as jnp
from jax.experimental import pallas as pl


def kernel(inputs, mask, W0, b0, w1, W2, b2, w3):
    raise NotImplementedError("write your pallas kernel here")


# speedup vs baseline: 3.5265x; 3.5265x over previous
import functools

import jax
import jax.numpy as jnp
from jax.experimental import pallas as pl


def _gelu(x):
    return 0.5 * x * (1.0 + jnp.tanh(0.7978845608028654 * (x + 0.044715 * x * x * x)))


def _answer_finder_kernel(x_ref, mc_ref, mr_ref, W0_ref, b0_ref, w1_ref,
                          W2_ref, b2_ref, w3_ref, out_ref):
    x = x_ref[0]
    mc = mc_ref[0]
    mr = mr_ref[0]
    W0 = W0_ref[...]
    b0 = b0_ref[...]
    w1 = w1_ref[...]
    W2 = W2_ref[...]
    b2 = b2_ref[...]
    w3 = w3_ref[...]

    h = _gelu(jnp.dot(x, W0, preferred_element_type=jnp.float32) + b0) * mc

    sl = jnp.sum(h * w1, axis=1, keepdims=True)
    slm = sl * mc + (1.0 - mc) * (-10.0)

    sc = _gelu(jnp.dot(h, W2, preferred_element_type=jnp.float32) + b2) * mc
    c = jnp.sum(sc * w3, axis=1, keepdims=True)
    a = jax.lax.dot_general(w3, h, (((1,), (1,)), ((), ())),
                            preferred_element_type=jnp.float32)

    m1 = jnp.max(slm)
    z1 = jnp.sum(jnp.exp(slm - m1))
    slp = (m1 + jnp.log(z1)) - slm

    s = out_ref.shape[1]
    ii = jax.lax.broadcasted_iota(jnp.int32, (s, s), 0)
    jj = jax.lax.broadcasted_iota(jnp.int32, (s, s), 1)
    valid = (mc > 0.0) & (mr > 0.0) & (jj >= ii)
    e = jnp.where(valid, a + c, -10.0)

    m2 = jnp.max(e)
    z2 = jnp.sum(jnp.exp(e - m2))
    out_ref[0] = slp + ((m2 + jnp.log(z2)) - e)


@jax.jit
def kernel(inputs, mask, W0, b0, w1, W2, b2, w3):
    B, S, D = inputs.shape
    U = W0.shape[1]
    mf = mask.astype(jnp.float32)
    mc = mf.reshape(B, S, 1)
    mr = mf.reshape(B, 1, S)
    grid_spec = pl.GridSpec(
        grid=(B,),
        in_specs=[
            pl.BlockSpec((1, S, D), lambda b: (b, 0, 0)),
            pl.BlockSpec((1, S, 1), lambda b: (b, 0, 0)),
            pl.BlockSpec((1, 1, S), lambda b: (b, 0, 0)),
            pl.BlockSpec((D, U), lambda b: (0, 0)),
            pl.BlockSpec((1, U), lambda b: (0, 0)),
            pl.BlockSpec((1, U), lambda b: (0, 0)),
            pl.BlockSpec((U, U), lambda b: (0, 0)),
            pl.BlockSpec((1, U), lambda b: (0, 0)),
            pl.BlockSpec((1, U), lambda b: (0, 0)),
        ],
        out_specs=pl.BlockSpec((1, S, S), lambda b: (b, 0, 0)),
    )
    return pl.pallas_call(
        _answer_finder_kernel,
        grid_spec=grid_spec,
        out_shape=jax.ShapeDtypeStruct((B, S, S), jnp.float32),
    )(inputs, mc, mr, W0, b0.reshape(1, U), w1.reshape(1, U),
      W2, b2.reshape(1, U), w3.reshape(1, U))


# speedup vs baseline: 3.5923x; 1.0187x over previous
import jax
import jax.numpy as jnp
from jax.experimental import pallas as pl


def _gelu(x):
    return 0.5 * x * (1.0 + jnp.tanh(0.7978845608028654 * (x + 0.044715 * x * x * x)))


def _answer_finder_kernel(x_ref, mr_ref, W0_ref, b0_ref, w1_ref,
                          W2_ref, b2_ref, w3_ref, out_ref):
    x = x_ref[0]
    mrowf = mr_ref[0].astype(jnp.float32)
    W0 = W0_ref[...]
    b0 = b0_ref[...]
    w1 = w1_ref[...]
    W2 = W2_ref[...]
    b2 = b2_ref[...]
    w3 = w3_ref[...]

    s = out_ref.shape[1]
    mcolf = jnp.transpose(mrowf, (1, 0))
    mcolb = mcolf > 0.0

    h = _gelu(jnp.dot(x, W0, preferred_element_type=jnp.float32) + b0)

    sl = jnp.sum(h * w1, axis=1, keepdims=True)
    slm = mcolf * sl + (mcolf - 1.0) * 10.0

    sc = _gelu(jnp.dot(h, W2, preferred_element_type=jnp.float32) + b2)
    c = jnp.sum(sc * w3, axis=1, keepdims=True)
    a_col = jnp.sum(h * w3, axis=1, keepdims=True)
    a_row = jax.lax.dot_general(w3, h, (((1,), (1,)), ((), ())),
                                preferred_element_type=jnp.float32)

    m1 = jnp.max(slm)
    z1 = jnp.sum(jnp.exp(slm - m1))
    slp = (m1 + jnp.log(z1)) - slm

    ii = jax.lax.broadcasted_iota(jnp.int32, (s, s), 0)
    jj = jax.lax.broadcasted_iota(jnp.int32, (s, s), 1)
    tri_f = jnp.where(jj >= ii, 1.0, 0.0)
    valid_f = tri_f * (mcolf * mrowf)

    neg = jnp.float32(-1e30)
    ma = jnp.max(jnp.where(mcolb, a_col, neg))
    mc = jnp.max(jnp.where(mcolb, c, neg))
    m2 = jnp.maximum(ma + mc, -10.0)
    ea = jnp.where(mcolb, jnp.exp(a_col - ma), 0.0)
    ec = jnp.where(mcolb, jnp.exp(c - mc), 0.0)
    rhs = jnp.concatenate([ea, mcolf], axis=1)
    suf = jax.lax.dot_general(tri_f, rhs, (((1,), (0,)), ((), ())),
                              preferred_element_type=jnp.float32)
    z2p = jnp.sum(suf[:, 0:1] * ec)
    npairs = jnp.sum(suf[:, 1:2] * mcolf)
    z2 = z2p * jnp.exp((ma + mc) - m2) + (s * s - npairs) * jnp.exp(-10.0 - m2)
    lse2 = m2 + jnp.log(z2)

    ut = slp + (lse2 + 10.0)
    d = c + 10.0
    out_ref[0] = ut - valid_f * (d + a_row)


@jax.jit
def kernel(inputs, mask, W0, b0, w1, W2, b2, w3):
    B, S, D = inputs.shape
    U = W0.shape[1]
    mr = mask.reshape(B, 1, S)
    grid_spec = pl.GridSpec(
        grid=(B,),
        in_specs=[
            pl.BlockSpec((1, S, D), lambda b: (b, 0, 0)),
            pl.BlockSpec((1, 1, S), lambda b: (b, 0, 0)),
            pl.BlockSpec((D, U), lambda b: (0, 0)),
            pl.BlockSpec((1, U), lambda b: (0, 0)),
            pl.BlockSpec((1, U), lambda b: (0, 0)),
            pl.BlockSpec((U, U), lambda b: (0, 0)),
            pl.BlockSpec((1, U), lambda b: (0, 0)),
            pl.BlockSpec((1, U), lambda b: (0, 0)),
        ],
        out_specs=pl.BlockSpec((1, S, S), lambda b: (b, 0, 0)),
    )
    return pl.pallas_call(
        _answer_finder_kernel,
        grid_spec=grid_spec,
        out_shape=jax.ShapeDtypeStruct((B, S, S), jnp.float32),
    )(inputs, mr, W0, b0.reshape(1, U), w1.reshape(1, U),
      W2, b2.reshape(1, U), w3.reshape(1, U))
